# Initial kernel scaffold; baseline (speedup 1.0000x reference)
import jax
import jax.numpy as jnp
from jax.experimental import pallas as pl


def kernel(entity_emb, user_emb, edge_index, edge_type, user_index, item_index, w):
    raise NotImplementedError("write your pallas kernel here")


# speedup vs baseline: 2.1721x; 2.1721x over previous
import functools

import numpy as np

import jax
import jax.numpy as jnp
from jax import lax
from jax.experimental import pallas as pl
from jax.experimental.pallas import tpu as pltpu
from jax.experimental.pallas import tpu_sc as plsc

NC = 2
NS = 16
NW = NC * NS
L = 16
D = 128
B = 80

_mesh = functools.partial(
    plsc.VectorSubcoreMesh, core_axis_name="c", subcore_axis_name="s")

f32 = jnp.float32
i32 = jnp.int32


def _zero_fill(buf, rows):
    def body(i, _):
        for j in range(D // L):
            buf[i, pl.ds(j * L, L)] = jnp.zeros((L,), f32)
        return _
    lax.fori_loop(0, rows, body, None)


def _zero_acc(acc, zbuf, tid, n_rows):
    zrows = zbuf.shape[0]
    per_tile = (n_rows // NS) // 8 * 8
    n_copies = per_tile // zrows
    for q in range(n_copies):
        pltpu.sync_copy(zbuf, acc.at[pl.ds(tid * per_tile + q * zrows, zrows)])
    rem = n_rows - per_tile * NS
    if rem:
        @pl.when(tid == 0)
        def _():
            pltpu.sync_copy(zbuf.at[pl.ds(0, rem)],
                            acc.at[pl.ds(per_tile * NS, rem)])


def _writeout(acc, out, cid, tid, n_rows):
    per_tile = (n_rows // NS) // 8 * 8
    pltpu.sync_copy(acc.at[pl.ds(tid * per_tile, per_tile)],
                    out.at[cid, pl.ds(tid * per_tile, per_tile)])
    rem = n_rows - per_tile * NS
    if rem:
        @pl.when(tid == 0)
        def _():
            pltpu.sync_copy(acc.at[pl.ds(per_tile * NS, rem)],
                            out.at[cid, pl.ds(per_tile * NS, rem)])


def _make_edge_pass(with_dot, E, N):
    per_w = E // NW
    n_chunks = per_w // B
    assert per_w % B == 0

    def body(dst_hbm, src_hbm, s_hbm, tab_hbm, *rest):
        if with_dot:
            (u_hbm, part_hbm, sout_hbm, idx_d, idx_s, s_v, s_new_v,
             rows_t, rows_u, mat, zbuf, acc, sem0, sem1) = rest
        else:
            (part_hbm, idx_d, idx_s, s_v,
             rows_t, zbuf, acc, sem0) = rest
        cid = lax.axis_index("c")
        tid = lax.axis_index("s")
        wid = cid * NS + tid

        _zero_fill(zbuf, zbuf.shape[0])
        _zero_acc(acc, zbuf, tid, N)
        plsc.subcore_barrier()

        def chunk(k, _):
            base = wid * per_w + k * B
            pltpu.sync_copy(dst_hbm.at[pl.ds(base, B)], idx_d)
            pltpu.sync_copy(src_hbm.at[pl.ds(base, B)], idx_s)
            pltpu.sync_copy(s_hbm.at[pl.ds(base, B)], s_v)
            cp_t = pltpu.async_copy(tab_hbm.at[idx_s], rows_t, sem0)
            if with_dot:
                cp_u = pltpu.async_copy(u_hbm.at[idx_d], rows_u, sem1)
            cp_t.wait()
            if with_dot:
                cp_u.wait()

            def group(g, _):
                s16 = s_v[pl.ds(g * L, L)]
                if with_dot:
                    lane = lax.iota(i32, L)
                    for e in range(L):
                        ei = g * L + e
                        vt = [rows_t[ei, pl.ds(j * L, L)]
                              for j in range(D // L)]
                        vu = [rows_u[ei, pl.ds(j * L, L)]
                              for j in range(D // L)]
                        acc_v = vt[0] * vu[0]
                        for j in range(1, D // L):
                            acc_v = acc_v + vt[j] * vu[j]
                        plsc.store_scatter(mat, [lane * (L + 1) + e], acc_v)
                    d16 = plsc.load_gather(mat, [lane])
                    for r in range(1, L):
                        d16 = d16 + plsc.load_gather(mat, [lane + r * (L + 1)])
                    coef16 = s16 * s16 * d16
                    s_new_v[pl.ds(g * L, L)] = coef16
                    for e in range(L):
                        ei = g * L + e
                        ce = coef16[e]
                        for j in range(D // L):
                            sl = pl.ds(j * L, L)
                            rows_t[ei, sl] = rows_t[ei, sl] * ce
                else:
                    for e in range(L):
                        ei = g * L + e
                        s_sc = s16[e]
                        for j in range(D // L):
                            sl = pl.ds(j * L, L)
                            rows_t[ei, sl] = rows_t[ei, sl] * s_sc
                return _

            lax.fori_loop(0, B // L, group, None)
            pltpu.sync_copy(rows_t, acc.at[idx_d], add=True)
            if with_dot:
                pltpu.sync_copy(s_new_v, sout_hbm.at[pl.ds(base, B)])
            return _

        lax.fori_loop(0, n_chunks, chunk, None)
        plsc.subcore_barrier()
        _writeout(acc, part_hbm, cid, tid, N)

    out_type = [jax.ShapeDtypeStruct((NC, N, D), f32)]
    scratch = [
        pltpu.VMEM((B,), i32),
        pltpu.VMEM((B,), i32),
        pltpu.VMEM((B,), f32),
    ]
    if with_dot:
        out_type.append(jax.ShapeDtypeStruct((E,), f32))
        scratch.append(pltpu.VMEM((B,), f32))
    scratch.append(pltpu.VMEM((B, D), f32))
    if with_dot:
        scratch.append(pltpu.VMEM((B, D), f32))
        scratch.append(pltpu.VMEM((L * (L + 1),), f32))
    scratch += [
        pltpu.VMEM((104, D), f32),
        pltpu.VMEM_SHARED((N, D), f32),
        pltpu.SemaphoreType.DMA,
    ]
    if with_dot:
        scratch.append(pltpu.SemaphoreType.DMA)

    return pl.kernel(body, out_type=out_type, mesh=_mesh(),
                     scratch_types=scratch,
                     compiler_params=pltpu.CompilerParams(
                         needs_layout_passes=False))


def _make_count_pass(E, NE, NU):
    per_w = E // NW
    n_chunks = per_w // B
    NEp = -(-NE // (NS * 640)) * NS * 640
    NUp = -(-NU // (NS * 640)) * NS * 640
    pe = NEp // NS
    pu = NUp // NS

    def body(cidx_hbm, uidx_hbm, pc_e_hbm, pc_u_hbm,
             idx1, idx2, ones_v, zb, acc_e, acc_u):
        cid = lax.axis_index("c")
        tid = lax.axis_index("s")
        wid = cid * NS + tid

        def zfill(i, _):
            zb[pl.ds(i * L, L)] = jnp.zeros((L,), f32)
            return _
        lax.fori_loop(0, zb.shape[0] // L, zfill, None)
        for g in range(B // L):
            ones_v[pl.ds(g * L, L)] = jnp.ones((L,), f32)
        for q in range(pe // 640):
            pltpu.sync_copy(zb, acc_e.at[pl.ds(tid * pe + q * 640, 640)])
        for q in range(pu // 640):
            pltpu.sync_copy(zb, acc_u.at[pl.ds(tid * pu + q * 640, 640)])
        plsc.subcore_barrier()

        def chunk(k, _):
            base = wid * per_w + k * B
            pltpu.sync_copy(cidx_hbm.at[pl.ds(base, B)], idx1)
            pltpu.sync_copy(uidx_hbm.at[pl.ds(base, B)], idx2)
            pltpu.sync_copy(ones_v, acc_e.at[idx1], add=True)
            pltpu.sync_copy(ones_v, acc_u.at[idx2], add=True)
            return _
        lax.fori_loop(0, n_chunks, chunk, None)
        plsc.subcore_barrier()
        pltpu.sync_copy(acc_e.at[pl.ds(tid * pe, pe)],
                        pc_e_hbm.at[cid, pl.ds(tid * pe, pe)])
        pltpu.sync_copy(acc_u.at[pl.ds(tid * pu, pu)],
                        pc_u_hbm.at[cid, pl.ds(tid * pu, pu)])

    out_type = [jax.ShapeDtypeStruct((NC, NEp), f32),
                jax.ShapeDtypeStruct((NC, NUp), f32)]
    scratch = [
        pltpu.VMEM((B,), i32),
        pltpu.VMEM((B,), i32),
        pltpu.VMEM((B,), f32),
        pltpu.VMEM((640,), f32),
        pltpu.VMEM_SHARED((NEp,), f32),
        pltpu.VMEM_SHARED((NUp,), f32),
    ]
    return pl.kernel(body, out_type=out_type, mesh=_mesh(),
                     scratch_types=scratch,
                     compiler_params=pltpu.CompilerParams(
                         needs_layout_passes=False))


def _combine(part, cnt, emb, squash):
    N = emb.shape[0]
    R = 1000

    def body(part_ref, cnt_ref, emb_ref, out_ref):
        p = part_ref[0] + part_ref[1]
        u = p / cnt_ref[...]
        if squash:
            n2 = jnp.sum(u * u, axis=1, keepdims=True)
            u = (n2 / (n2 + 1.0)) * u / jnp.maximum(jnp.sqrt(n2), 1e-12)
        out_ref[...] = u + emb_ref[...]

    return pl.pallas_call(
        body,
        grid=(N // R,),
        in_specs=[
            pl.BlockSpec((NC, R, D), lambda i: (0, i, 0)),
            pl.BlockSpec((R, 1), lambda i: (i, 0)),
            pl.BlockSpec((R, D), lambda i: (i, 0)),
        ],
        out_specs=pl.BlockSpec((R, D), lambda i: (i, 0)),
        out_shape=jax.ShapeDtypeStruct((N, D), f32),
    )(part, cnt, emb)


def _wsum3(u0, u1, u2, aw):
    N = u0.shape[0]
    R = 1000

    def body(a_ref, r0, r1, r2, out_ref):
        out_ref[...] = (a_ref[0] * r0[...] + a_ref[1] * r1[...]
                        + a_ref[2] * r2[...])

    blk = pl.BlockSpec((R, D), lambda i: (i, 0))
    return pl.pallas_call(
        body,
        grid=(N // R,),
        in_specs=[pl.BlockSpec(memory_space=pltpu.MemorySpace.SMEM),
                  blk, blk, blk],
        out_specs=blk,
        out_shape=jax.ShapeDtypeStruct((N, D), f32),
    )(aw, u0, u1, u2)


def kernel(entity_emb, user_emb, edge_index, edge_type, user_index,
           item_index, w):
    n_ent = entity_emb.shape[0]
    n_user = user_emb.shape[0]
    E = edge_index.shape[1]
    nnz = user_index.shape[0]
    head = edge_index[0]
    tail = edge_index[1]

    pass0_e = _make_edge_pass(False, E, n_ent)
    passn_e = _make_edge_pass(True, E, n_ent)
    pass0_u = _make_edge_pass(False, nnz, n_user)
    passn_u = _make_edge_pass(True, nnz, n_user)
    count_p = _make_count_pass(E, 3 * n_ent, n_user)

    cidx = edge_type * n_ent + head
    pc_e, pc_u = count_p(cidx, user_index)
    cnt_ent = jnp.maximum((pc_e[0] + pc_e[1])[:3 * n_ent], 1.0)
    cnt_ent = cnt_ent.reshape(3, n_ent)
    cnt_user = jnp.maximum((pc_u[0] + pc_u[1])[:n_user], 1.0)[:, None]

    ent_list = []
    for i in range(3):
        mask = (edge_type == i).astype(f32)
        cnt_i = cnt_ent[i][:, None]
        (part,) = pass0_e(head, tail, mask, entity_emb)
        u = _combine(part, cnt_i, entity_emb, True)
        part, s1 = passn_e(head, tail, mask, entity_emb, u)
        u = _combine(part, cnt_i, entity_emb, True)
        part, _s2 = passn_e(head, tail, s1, entity_emb, u)
        ent_list.append(_combine(part, cnt_i, entity_emb, False))

    ew = jnp.exp(w)
    aw = ew / jnp.sum(ew)
    entity_agg = _wsum3(ent_list[0], ent_list[1], ent_list[2], aw)

    ones = jnp.ones((nnz,), f32)
    (part,) = pass0_u(user_index, item_index, ones, entity_emb)
    u = _combine(part, cnt_user, user_emb, True)
    part, _ = passn_u(user_index, item_index, ones, entity_emb, u)
    u = _combine(part, cnt_user, user_emb, True)
    part, _ = passn_u(user_index, item_index, ones, entity_emb, u)
    user_agg = _combine(part, cnt_user, user_emb, False)

    return entity_agg, user_agg


# speedup vs baseline: 3.0245x; 1.3924x over previous
import functools

import numpy as np

import jax
import jax.numpy as jnp
from jax import lax
from jax.experimental import pallas as pl
from jax.experimental.pallas import tpu as pltpu
from jax.experimental.pallas import tpu_sc as plsc

NC = 2
NS = 16
NW = NC * NS
L = 16
D = 128
B = 80

_mesh = functools.partial(
    plsc.VectorSubcoreMesh, core_axis_name="c", subcore_axis_name="s")

f32 = jnp.float32
i32 = jnp.int32


def _zero_fill(buf, rows):
    def body(i, _):
        for j in range(D // L):
            buf[i, pl.ds(j * L, L)] = jnp.zeros((L,), f32)
        return _
    lax.fori_loop(0, rows, body, None)


def _zero_acc(acc, zbuf, tid, n_rows):
    zrows = zbuf.shape[0]
    per_tile = (n_rows // NS) // 8 * 8
    n_copies = per_tile // zrows
    for q in range(n_copies):
        pltpu.sync_copy(zbuf, acc.at[pl.ds(tid * per_tile + q * zrows, zrows)])
    tail = per_tile - n_copies * zrows
    if tail:
        pltpu.sync_copy(zbuf.at[pl.ds(0, tail)],
                        acc.at[pl.ds(tid * per_tile + n_copies * zrows, tail)])
    rem = n_rows - per_tile * NS
    if rem:
        @pl.when(tid == 0)
        def _():
            pltpu.sync_copy(zbuf.at[pl.ds(0, rem)],
                            acc.at[pl.ds(per_tile * NS, rem)])


def _writeout(acc, out, cid, tid, n_rows):
    per_tile = (n_rows // NS) // 8 * 8
    pltpu.sync_copy(acc.at[pl.ds(tid * per_tile, per_tile)],
                    out.at[cid, pl.ds(tid * per_tile, per_tile)])
    rem = n_rows - per_tile * NS
    if rem:
        @pl.when(tid == 0)
        def _():
            pltpu.sync_copy(acc.at[pl.ds(per_tile * NS, rem)],
                            out.at[cid, pl.ds(per_tile * NS, rem)])


def _make_edge_pass(with_dot, E, N):
    per_w = E // NW
    n_chunks = per_w // B
    assert per_w % B == 0
    assert n_chunks >= 3 and n_chunks % 2 == 1

    def body(dst_hbm, src_hbm, s_hbm, tab_hbm, *rest):
        if with_dot:
            (u_hbm, part_hbm, sout_hbm, idx_d, idx_s, s_v, s_new,
             rows_t, rows_u, mat, acc, gt, gu, sc, st, sw) = rest
        else:
            (part_hbm, idx_d, idx_s, s_v,
             rows_t, acc, gt, sc, st) = rest
        cid = lax.axis_index("c")
        tid = lax.axis_index("s")
        wid = cid * NS + tid

        _zero_fill(rows_t[0], B)
        _zero_acc(acc, rows_t[0], tid, N)

        def stage(k, p):
            pltpu.async_copy(dst_hbm.at[wid, k], idx_d[p], st[p])
            pltpu.async_copy(src_hbm.at[wid, k], idx_s[p], st[p])
            pltpu.async_copy(s_hbm.at[wid, k], s_v[p], st[p])

        def wait_stage(p):
            pltpu.make_async_copy(dst_hbm.at[wid, 0], idx_d[p], st[p]).wait()
            pltpu.make_async_copy(src_hbm.at[wid, 0], idx_s[p], st[p]).wait()
            pltpu.make_async_copy(s_hbm.at[wid, 0], s_v[p], st[p]).wait()

        def issue_gather(p):
            pltpu.async_copy(tab_hbm.at[idx_s[p]], rows_t[p], gt[p])
            if with_dot:
                pltpu.async_copy(u_hbm.at[idx_d[p]], rows_u[p], gu[p])

        def wait_gather(p):
            pltpu.make_async_copy(
                tab_hbm.at[idx_s[p]], rows_t[p], gt[p]).wait()
            if with_dot:
                pltpu.make_async_copy(
                    u_hbm.at[idx_d[p]], rows_u[p], gu[p]).wait()

        def issue_scatter(k, p):
            pltpu.async_copy(rows_t[p], acc.at[idx_d[p]], sc[p], add=True)
            if with_dot:
                pltpu.async_copy(s_new[p], sout_hbm.at[wid, k], sw[p])

        def wait_scatter(p):
            pltpu.make_async_copy(rows_t[p], acc.at[idx_d[p]], sc[p]).wait()
            if with_dot:
                pltpu.make_async_copy(
                    s_new[p], sout_hbm.at[wid, 0], sw[p]).wait()

        def compute(p):
            rt = rows_t[p]
            if with_dot:
                ru = rows_u[p]

            def group(g, _):
                s16 = s_v[p][pl.ds(g * L, L)]
                if with_dot:
                    lane = lax.iota(i32, L)
                    for e in range(L):
                        ei = g * L + e
                        vt = [rt[ei, pl.ds(j * L, L)] for j in range(D // L)]
                        vu = [ru[ei, pl.ds(j * L, L)] for j in range(D // L)]
                        acc_v = vt[0] * vu[0]
                        for j in range(1, D // L):
                            acc_v = acc_v + vt[j] * vu[j]
                        plsc.store_scatter(mat, [lane * (L + 1) + e], acc_v)
                    d16 = plsc.load_gather(mat, [lane])
                    for r in range(1, L):
                        d16 = d16 + plsc.load_gather(mat,
                                                     [lane + r * (L + 1)])
                    coef16 = s16 * s16 * d16
                    s_new[p][pl.ds(g * L, L)] = coef16
                    for e in range(L):
                        ei = g * L + e
                        ce = coef16[e]
                        for j in range(D // L):
                            sl = pl.ds(j * L, L)
                            rt[ei, sl] = rt[ei, sl] * ce
                else:
                    for e in range(L):
                        ei = g * L + e
                        s_sc = s16[e]
                        for j in range(D // L):
                            sl = pl.ds(j * L, L)
                            rt[ei, sl] = rt[ei, sl] * s_sc
                return _

            lax.fori_loop(0, B // L, group, None)

        def step(k, p, first=False):
            wait_gather(p)
            compute(p)
            issue_scatter(k, p)
            q = 1 - p
            if not first:
                wait_scatter(q)

            @pl.when(k + 1 < n_chunks)
            def _():
                stage(k + 1, q)
                wait_stage(q)
                issue_gather(q)

        stage(0, 0)
        wait_stage(0)
        issue_gather(0)
        step(0, 0, first=True)

        def pair(k2, _):
            k = 1 + 2 * k2
            step(k, 1)
            step(k + 1, 0)
            return _

        lax.fori_loop(0, (n_chunks - 1) // 2, pair, None)
        wait_scatter((n_chunks - 1) % 2)

        plsc.subcore_barrier()
        _writeout(acc, part_hbm, cid, tid, N)

    out_type = [jax.ShapeDtypeStruct((NC, N, D), f32)]
    scratch = [
        [pltpu.VMEM((B,), i32)] * 2,
        [pltpu.VMEM((B,), i32)] * 2,
        [pltpu.VMEM((B,), f32)] * 2,
    ]
    if with_dot:
        out_type.append(jax.ShapeDtypeStruct((NW, n_chunks, B), f32))
        scratch.append([pltpu.VMEM((B,), f32)] * 2)
    scratch.append([pltpu.VMEM((B, D), f32)] * 2)
    if with_dot:
        scratch.append([pltpu.VMEM((B, D), f32)] * 2)
        scratch.append(pltpu.VMEM((L * (L + 1),), f32))
    scratch += [
        pltpu.VMEM_SHARED((N, D), f32),
        [pltpu.SemaphoreType.DMA] * 2,
    ]
    if with_dot:
        scratch.append([pltpu.SemaphoreType.DMA] * 2)
    scratch.append([pltpu.SemaphoreType.DMA] * 2)
    scratch.append([pltpu.SemaphoreType.DMA] * 2)
    if with_dot:
        scratch.append([pltpu.SemaphoreType.DMA] * 2)

    return pl.kernel(body, out_type=out_type, mesh=_mesh(),
                     scratch_types=scratch,
                     compiler_params=pltpu.CompilerParams(
                         needs_layout_passes=False))


def _make_count_pass(E, NE, NU):
    per_w = E // NW
    n_chunks = per_w // B
    NEp = -(-NE // (NS * 640)) * NS * 640
    NUp = -(-NU // (NS * 640)) * NS * 640
    pe = NEp // NS
    pu = NUp // NS

    def body(cidx_hbm, uidx_hbm, pc_e_hbm, pc_u_hbm,
             cidx_all, uidx_all, ones_v, zb, acc_e, acc_u, sem_e, sem_u):
        cid = lax.axis_index("c")
        tid = lax.axis_index("s")
        wid = cid * NS + tid

        def zfill(i, _):
            zb[pl.ds(i * L, L)] = jnp.zeros((L,), f32)
            return _
        lax.fori_loop(0, zb.shape[0] // L, zfill, None)
        for g in range(B // L):
            ones_v[pl.ds(g * L, L)] = jnp.ones((L,), f32)
        for q in range(pe // 640):
            pltpu.sync_copy(zb, acc_e.at[pl.ds(tid * pe + q * 640, 640)])
        for q in range(pu // 640):
            pltpu.sync_copy(zb, acc_u.at[pl.ds(tid * pu + q * 640, 640)])
        pltpu.sync_copy(cidx_hbm.at[wid], cidx_all)
        pltpu.sync_copy(uidx_hbm.at[wid], uidx_all)
        plsc.subcore_barrier()

        def chunk(k, _):
            pltpu.async_copy(ones_v, acc_e.at[cidx_all.at[k]], sem_e,
                             add=True)
            pltpu.async_copy(ones_v, acc_u.at[uidx_all.at[k]], sem_u,
                             add=True)
            return _
        lax.fori_loop(0, n_chunks, chunk, None)

        def drain(k, _):
            pltpu.make_async_copy(ones_v, acc_e.at[cidx_all.at[0]],
                                  sem_e).wait()
            pltpu.make_async_copy(ones_v, acc_u.at[uidx_all.at[0]],
                                  sem_u).wait()
            return _
        lax.fori_loop(0, n_chunks, drain, None)
        plsc.subcore_barrier()
        pltpu.sync_copy(acc_e.at[pl.ds(tid * pe, pe)],
                        pc_e_hbm.at[cid, pl.ds(tid * pe, pe)])
        pltpu.sync_copy(acc_u.at[pl.ds(tid * pu, pu)],
                        pc_u_hbm.at[cid, pl.ds(tid * pu, pu)])

    out_type = [jax.ShapeDtypeStruct((NC, NEp), f32),
                jax.ShapeDtypeStruct((NC, NUp), f32)]
    scratch = [
        pltpu.VMEM((n_chunks, B), i32),
        pltpu.VMEM((n_chunks, B), i32),
        pltpu.VMEM((B,), f32),
        pltpu.VMEM((640,), f32),
        pltpu.VMEM_SHARED((NEp,), f32),
        pltpu.VMEM_SHARED((NUp,), f32),
        pltpu.SemaphoreType.DMA,
        pltpu.SemaphoreType.DMA,
    ]
    return pl.kernel(body, out_type=out_type, mesh=_mesh(),
                     scratch_types=scratch,
                     compiler_params=pltpu.CompilerParams(
                         needs_layout_passes=False))


def _combine(part, cnt, emb, squash):
    N = emb.shape[0]
    R = 1000

    def body(part_ref, cnt_ref, emb_ref, out_ref):
        p = part_ref[0] + part_ref[1]
        u = p / cnt_ref[...]
        if squash:
            n2 = jnp.sum(u * u, axis=1, keepdims=True)
            u = (n2 / (n2 + 1.0)) * u / jnp.maximum(jnp.sqrt(n2), 1e-12)
        out_ref[...] = u + emb_ref[...]

    return pl.pallas_call(
        body,
        grid=(N // R,),
        in_specs=[
            pl.BlockSpec((NC, R, D), lambda i: (0, i, 0)),
            pl.BlockSpec((R, 1), lambda i: (i, 0)),
            pl.BlockSpec((R, D), lambda i: (i, 0)),
        ],
        out_specs=pl.BlockSpec((R, D), lambda i: (i, 0)),
        out_shape=jax.ShapeDtypeStruct((N, D), f32),
    )(part, cnt, emb)


def _wsum3(u0, u1, u2, aw):
    N = u0.shape[0]
    R = 1000

    def body(a_ref, r0, r1, r2, out_ref):
        out_ref[...] = (a_ref[0] * r0[...] + a_ref[1] * r1[...]
                        + a_ref[2] * r2[...])

    blk = pl.BlockSpec((R, D), lambda i: (i, 0))
    return pl.pallas_call(
        body,
        grid=(N // R,),
        in_specs=[pl.BlockSpec(memory_space=pltpu.MemorySpace.SMEM),
                  blk, blk, blk],
        out_specs=blk,
        out_shape=jax.ShapeDtypeStruct((N, D), f32),
    )(aw, u0, u1, u2)


def kernel(entity_emb, user_emb, edge_index, edge_type, user_index,
           item_index, w):
    n_ent = entity_emb.shape[0]
    n_user = user_emb.shape[0]
    E = edge_index.shape[1]
    nnz = user_index.shape[0]
    head = edge_index[0]
    tail = edge_index[1]

    pass0_e = _make_edge_pass(False, E, n_ent)
    passn_e = _make_edge_pass(True, E, n_ent)
    pass0_u = _make_edge_pass(False, nnz, n_user)
    passn_u = _make_edge_pass(True, nnz, n_user)
    count_p = _make_count_pass(E, 3 * n_ent, n_user)

    def shp(a):
        return a.reshape(NW, -1, B)

    cidx = shp(edge_type * n_ent + head)
    head3, tail3 = shp(head), shp(tail)
    ui3, ii3 = shp(user_index), shp(item_index)
    pc_e, pc_u = count_p(cidx, ui3)
    cnt_ent = jnp.maximum((pc_e[0] + pc_e[1])[:3 * n_ent], 1.0)
    cnt_ent = cnt_ent.reshape(3, n_ent)
    cnt_user = jnp.maximum((pc_u[0] + pc_u[1])[:n_user], 1.0)[:, None]

    ent_list = []
    for i in range(3):
        mask = shp((edge_type == i).astype(f32))
        cnt_i = cnt_ent[i][:, None]
        (part,) = pass0_e(head3, tail3, mask, entity_emb)
        u = _combine(part, cnt_i, entity_emb, True)
        part, s1 = passn_e(head3, tail3, mask, entity_emb, u)
        u = _combine(part, cnt_i, entity_emb, True)
        part, _s2 = passn_e(head3, tail3, s1, entity_emb, u)
        ent_list.append(_combine(part, cnt_i, entity_emb, False))

    ew = jnp.exp(w)
    aw = ew / jnp.sum(ew)
    entity_agg = _wsum3(ent_list[0], ent_list[1], ent_list[2], aw)

    ones = jnp.ones(ui3.shape, f32)
    (part,) = pass0_u(ui3, ii3, ones, entity_emb)
    u = _combine(part, cnt_user, user_emb, True)
    part, _ = passn_u(ui3, ii3, ones, entity_emb, u)
    u = _combine(part, cnt_user, user_emb, True)
    part, _ = passn_u(ui3, ii3, ones, entity_emb, u)
    user_agg = _combine(part, cnt_user, user_emb, False)

    return entity_agg, user_agg
